# Initial kernel scaffold; baseline (speedup 1.0000x reference)
import jax
import jax.numpy as jnp
from jax.experimental import pallas as pl


def kernel(x, edge_index, W1, b1, W2, b2, W3, b3, W4, b4):
    raise NotImplementedError("write your pallas kernel here")


# speedup vs baseline: 10.9376x; 10.9376x over previous
import functools

import jax
import jax.numpy as jnp
from jax import lax
from jax.experimental import pallas as pl
from jax.experimental.pallas import tpu as pltpu
from jax.experimental.pallas import tpu_sc as plsc

N = 10000
E = 320000
NPAD = 10240
NC, NS = 2, 16
NW = NC * NS
CH = 128
EPW = 10240
NCHUNK = EPW // CH
RPT = NPAD // NS
PAD_DST = NPAD - 8

_MESH = dict(core_axis_name="c", subcore_axis_name="s")


def _deg_body(dst_hbm, deg_hbm, dstv, onesv, stripev, degsp):
    c = lax.axis_index("c")
    s = lax.axis_index("s")
    wid = c * NS + s

    def zero_body(i, _):
        stripev[pl.ds(i * 16, 16)] = jnp.zeros((16,), jnp.float32)
        return 0

    lax.fori_loop(0, RPT // 16, zero_body, 0)

    def ones_body(i, _):
        onesv[pl.ds(i * 16, 16)] = jnp.ones((16,), jnp.float32)
        return 0

    lax.fori_loop(0, CH // 16, ones_body, 0)

    pltpu.sync_copy(stripev, degsp.at[pl.ds(s * RPT, RPT)])
    pltpu.sync_copy(dst_hbm.at[wid], dstv)
    plsc.subcore_barrier()

    def body(i, _):
        pltpu.sync_copy(onesv, degsp.at[dstv.at[i]], add=True)
        return 0

    lax.fori_loop(0, NCHUNK, body, 0)
    plsc.subcore_barrier()

    pltpu.sync_copy(degsp.at[pl.ds(s * RPT, RPT)], stripev)
    pltpu.sync_copy(stripev, deg_hbm.at[c, pl.ds(s * RPT, RPT)])


_SC_PARAMS = pltpu.CompilerParams(use_tc_tiling_on_sc=False)

_deg_call = pl.kernel(
    _deg_body,
    out_type=jax.ShapeDtypeStruct((NC, NPAD), jnp.float32),
    mesh=plsc.VectorSubcoreMesh(**_MESH),
    compiler_params=_SC_PARAMS,
    scratch_types=[
        pltpu.VMEM((NCHUNK, CH), jnp.int32),
        pltpu.VMEM((CH,), jnp.float32),
        pltpu.VMEM((RPT,), jnp.float32),
        pltpu.VMEM_SHARED((NPAD,), jnp.float32),
    ],
)


def _agg_body(y_hbm, src_hbm, dst_hbm, out_hbm, srcv, dstv, bufv, stripev, tsp,
              sem):
    c = lax.axis_index("c")
    s = lax.axis_index("s")
    wid = c * NS + s

    pltpu.sync_copy(y_hbm.at[pl.ds(s * RPT, RPT)], stripev)
    pltpu.sync_copy(stripev, tsp.at[pl.ds(s * RPT, RPT)])
    pltpu.sync_copy(src_hbm.at[wid], srcv)
    pltpu.sync_copy(dst_hbm.at[wid], dstv)
    plsc.subcore_barrier()

    def body(i, _):
        pltpu.async_copy(y_hbm.at[srcv.at[i]], bufv, sem).wait()
        pltpu.sync_copy(bufv, tsp.at[dstv.at[i]], add=True)
        return 0

    lax.fori_loop(0, NCHUNK, body, 0)
    plsc.subcore_barrier()

    pltpu.sync_copy(tsp.at[pl.ds(s * RPT, RPT)], stripev)
    pltpu.sync_copy(stripev, out_hbm.at[c, pl.ds(s * RPT, RPT)])


def _make_agg(d):
    return pl.kernel(
        _agg_body,
        out_type=jax.ShapeDtypeStruct((NC, NPAD, d), jnp.float32),
        mesh=plsc.VectorSubcoreMesh(**_MESH),
        compiler_params=_SC_PARAMS,
        scratch_types=[
            pltpu.VMEM((NCHUNK, CH), jnp.int32),
            pltpu.VMEM((NCHUNK, CH), jnp.int32),
            pltpu.VMEM((CH, d), jnp.float32),
            pltpu.VMEM((RPT, d), jnp.float32),
            pltpu.VMEM_SHARED((NPAD, d), jnp.float32),
            pltpu.SemaphoreType.DMA,
        ],
    )


_agg64 = _make_agg(64)
_agg32 = _make_agg(32)


_BLK = 512
_NBLK = NPAD // _BLK


def _tc_first_body(x_ref, w_ref, d0_ref, d1_ref, y_ref, dinv_ref):
    dinv = lax.rsqrt(1.0 + d0_ref[...] + d1_ref[...])
    y_ref[...] = jnp.dot(x_ref[...], w_ref[...],
                         preferred_element_type=jnp.float32) * dinv
    dinv_ref[...] = dinv


def _tc_first(x_p, w1, d0, d1):
    return pl.pallas_call(
        _tc_first_body,
        grid=(_NBLK,),
        in_specs=[
            pl.BlockSpec((_BLK, 128), lambda i: (i, 0)),
            pl.BlockSpec((128, 64), lambda i: (0, 0)),
            pl.BlockSpec((_BLK, 1), lambda i: (i, 0)),
            pl.BlockSpec((_BLK, 1), lambda i: (i, 0)),
        ],
        out_specs=[
            pl.BlockSpec((_BLK, 64), lambda i: (i, 0)),
            pl.BlockSpec((_BLK, 1), lambda i: (i, 0)),
        ],
        out_shape=[
            jax.ShapeDtypeStruct((NPAD, 64), jnp.float32),
            jax.ShapeDtypeStruct((NPAD, 1), jnp.float32),
        ],
    )(x_p, w1, d0, d1)


def _tc_mid_body(t0_ref, t1_ref, y_ref, dinv_ref, b_ref, w_ref, out_ref):
    dinv = dinv_ref[...]
    h = jnp.maximum(dinv * (t0_ref[...] + t1_ref[...] - y_ref[...]) + b_ref[...],
                    0.0)
    out_ref[...] = jnp.dot(h, w_ref[...],
                           preferred_element_type=jnp.float32) * dinv


def _tc_mid(t0, t1, y, dinv, b, w, d_in, d_out):
    return pl.pallas_call(
        _tc_mid_body,
        grid=(_NBLK,),
        in_specs=[
            pl.BlockSpec((_BLK, d_in), lambda i: (i, 0)),
            pl.BlockSpec((_BLK, d_in), lambda i: (i, 0)),
            pl.BlockSpec((_BLK, d_in), lambda i: (i, 0)),
            pl.BlockSpec((_BLK, 1), lambda i: (i, 0)),
            pl.BlockSpec((1, d_in), lambda i: (0, 0)),
            pl.BlockSpec((d_in, d_out), lambda i: (0, 0)),
        ],
        out_specs=pl.BlockSpec((_BLK, d_out), lambda i: (i, 0)),
        out_shape=jax.ShapeDtypeStruct((NPAD, d_out), jnp.float32),
    )(t0, t1, y, dinv, b, w)


def _tc_last_body(t0_ref, t1_ref, y_ref, dinv_ref, b_ref, out_ref):
    out_ref[...] = (dinv_ref[...] * (t0_ref[...] + t1_ref[...] - y_ref[...])
                    + b_ref[...])


def _tc_last(t0, t1, y, dinv, b, d):
    return pl.pallas_call(
        _tc_last_body,
        grid=(_NBLK,),
        in_specs=[
            pl.BlockSpec((_BLK, d), lambda i: (i, 0)),
            pl.BlockSpec((_BLK, d), lambda i: (i, 0)),
            pl.BlockSpec((_BLK, d), lambda i: (i, 0)),
            pl.BlockSpec((_BLK, 1), lambda i: (i, 0)),
            pl.BlockSpec((1, d), lambda i: (0, 0)),
        ],
        out_specs=pl.BlockSpec((_BLK, d), lambda i: (i, 0)),
        out_shape=jax.ShapeDtypeStruct((NPAD, d), jnp.float32),
    )(t0, t1, y, dinv, b)


def kernel(x, edge_index, W1, b1, W2, b2, W3, b3, W4, b4):
    src = edge_index[0].astype(jnp.int32)
    dst = edge_index[1].astype(jnp.int32)
    npad_e = NW * EPW - E
    src3 = jnp.concatenate(
        [src, jnp.zeros((npad_e,), jnp.int32)]).reshape(NW, NCHUNK, CH)
    dst3 = jnp.concatenate(
        [dst, jnp.full((npad_e,), PAD_DST, jnp.int32)]).reshape(NW, NCHUNK, CH)

    x_p = jnp.pad(x, ((0, NPAD - N), (0, 0)))

    deg = _deg_call(dst3)
    d0 = deg[0][:, None]
    d1 = deg[1][:, None]

    y1, dinv = _tc_first(x_p, W1, d0, d1)

    t = _agg64(y1, src3, dst3)
    y2 = _tc_mid(t[0], t[1], y1, dinv, b1[None, :], W2, 64, 64)
    t = _agg64(y2, src3, dst3)
    y3 = _tc_mid(t[0], t[1], y2, dinv, b2[None, :], W3, 64, 64)
    t = _agg64(y3, src3, dst3)
    y4 = _tc_mid(t[0], t[1], y3, dinv, b3[None, :], W4, 64, 32)
    t = _agg32(y4, src3, dst3)
    out = _tc_last(t[0], t[1], y4, dinv, b4[None, :], 32)

    return out[:N]


# speedup vs baseline: 12.5645x; 1.1487x over previous
import functools

import jax
import jax.numpy as jnp
from jax import lax
from jax.experimental import pallas as pl
from jax.experimental.pallas import tpu as pltpu
from jax.experimental.pallas import tpu_sc as plsc

N = 10000
E = 320000
NPAD = 10240
NC, NS = 2, 16
NW = NC * NS
CH = 128
EPW = 10240
NCHUNK = EPW // CH
RPT = NPAD // NS
PAD_DST = NPAD - 8

_MESH = dict(core_axis_name="c", subcore_axis_name="s")


def _deg_body(dst_hbm, deg_hbm, dstv, onesv, stripev, degsp):
    c = lax.axis_index("c")
    s = lax.axis_index("s")
    wid = c * NS + s

    def zero_body(i, _):
        stripev[pl.ds(i * 16, 16)] = jnp.zeros((16,), jnp.float32)
        return 0

    lax.fori_loop(0, RPT // 16, zero_body, 0)

    def ones_body(i, _):
        onesv[pl.ds(i * 16, 16)] = jnp.ones((16,), jnp.float32)
        return 0

    lax.fori_loop(0, CH // 16, ones_body, 0)

    pltpu.sync_copy(stripev, degsp.at[pl.ds(s * RPT, RPT)])
    pltpu.sync_copy(dst_hbm.at[wid], dstv)
    plsc.subcore_barrier()

    def body(i, _):
        pltpu.sync_copy(onesv, degsp.at[dstv.at[i]], add=True)
        return 0

    lax.fori_loop(0, NCHUNK, body, 0)
    plsc.subcore_barrier()

    pltpu.sync_copy(degsp.at[pl.ds(s * RPT, RPT)], stripev)
    pltpu.sync_copy(stripev, deg_hbm.at[c, pl.ds(s * RPT, RPT)])


_SC_PARAMS = pltpu.CompilerParams(use_tc_tiling_on_sc=False)

_deg_call = pl.kernel(
    _deg_body,
    out_type=jax.ShapeDtypeStruct((NC, NPAD), jnp.float32),
    mesh=plsc.VectorSubcoreMesh(**_MESH),
    compiler_params=_SC_PARAMS,
    scratch_types=[
        pltpu.VMEM((NCHUNK, CH), jnp.int32),
        pltpu.VMEM((CH,), jnp.float32),
        pltpu.VMEM((RPT,), jnp.float32),
        pltpu.VMEM_SHARED((NPAD,), jnp.float32),
    ],
)


_NBUF = 4


def _agg_body(y_hbm, src_hbm, dst_hbm, out_hbm, srcv, dstv,
              buf0, buf1, buf2, buf3, tsp,
              g0, g1, g2, g3, s0, s1, s2, s3):
    c = lax.axis_index("c")
    s = lax.axis_index("s")
    wid = c * NS + s
    bufs = (buf0, buf1, buf2, buf3)
    gsem = (g0, g1, g2, g3)
    ssem = (s0, s1, s2, s3)

    pltpu.sync_copy(y_hbm.at[pl.ds(s * RPT, RPT)], tsp.at[pl.ds(s * RPT, RPT)])
    pltpu.sync_copy(src_hbm.at[wid], srcv)
    pltpu.sync_copy(dst_hbm.at[wid], dstv)
    plsc.subcore_barrier()

    for j in range(_NBUF):
        pltpu.async_copy(y_hbm.at[srcv.at[j]], bufs[j], gsem[j])

    def body(i, _):
        scat = []
        for j in range(_NBUF):
            cch = _NBUF * i + j
            pltpu.make_async_copy(y_hbm.at[srcv.at[cch]], bufs[j],
                                  gsem[j]).wait()
            scat.append(pltpu.async_copy(bufs[j], tsp.at[dstv.at[cch]],
                                         ssem[j], add=True))
        for j in range(_NBUF):
            nch = _NBUF * (i + 1) + j
            scat[j].wait()
            pltpu.async_copy(y_hbm.at[srcv.at[nch]], bufs[j], gsem[j])
        return 0

    lax.fori_loop(0, NCHUNK // _NBUF - 1, body, 0)
    for j in range(_NBUF):
        cch = NCHUNK - _NBUF + j
        pltpu.make_async_copy(y_hbm.at[srcv.at[cch]], bufs[j], gsem[j]).wait()
        pltpu.sync_copy(bufs[j], tsp.at[dstv.at[cch]], add=True)
    plsc.subcore_barrier()

    pltpu.sync_copy(tsp.at[pl.ds(s * RPT, RPT)],
                    out_hbm.at[c, pl.ds(s * RPT, RPT)])


def _make_agg(d):
    return pl.kernel(
        _agg_body,
        out_type=jax.ShapeDtypeStruct((NC, NPAD, d), jnp.float32),
        mesh=plsc.VectorSubcoreMesh(**_MESH),
        compiler_params=_SC_PARAMS,
        scratch_types=[
            pltpu.VMEM((NCHUNK, CH), jnp.int32),
            pltpu.VMEM((NCHUNK, CH), jnp.int32),
        ] + [pltpu.VMEM((CH, d), jnp.float32) for _ in range(_NBUF)] + [
            pltpu.VMEM_SHARED((NPAD, d), jnp.float32),
        ] + [pltpu.SemaphoreType.DMA for _ in range(2 * _NBUF)],
    )


_agg64 = _make_agg(64)
_agg32 = _make_agg(32)


_BLK = 512
_NBLK = NPAD // _BLK


def _tc_first_body(x_ref, w_ref, d0_ref, d1_ref, y_ref, dinv_ref):
    dinv = lax.rsqrt(1.0 + d0_ref[...] + d1_ref[...])
    y_ref[...] = jnp.dot(x_ref[...], w_ref[...],
                         preferred_element_type=jnp.float32) * dinv
    dinv_ref[...] = dinv


def _tc_first(x_p, w1, d0, d1):
    return pl.pallas_call(
        _tc_first_body,
        grid=(_NBLK,),
        in_specs=[
            pl.BlockSpec((_BLK, 128), lambda i: (i, 0)),
            pl.BlockSpec((128, 64), lambda i: (0, 0)),
            pl.BlockSpec((_BLK, 1), lambda i: (i, 0)),
            pl.BlockSpec((_BLK, 1), lambda i: (i, 0)),
        ],
        out_specs=[
            pl.BlockSpec((_BLK, 64), lambda i: (i, 0)),
            pl.BlockSpec((_BLK, 1), lambda i: (i, 0)),
        ],
        out_shape=[
            jax.ShapeDtypeStruct((NPAD, 64), jnp.float32),
            jax.ShapeDtypeStruct((NPAD, 1), jnp.float32),
        ],
    )(x_p, w1, d0, d1)


def _tc_mid_body(t0_ref, t1_ref, y_ref, dinv_ref, b_ref, w_ref, out_ref):
    dinv = dinv_ref[...]
    h = jnp.maximum(dinv * (t0_ref[...] + t1_ref[...] - y_ref[...]) + b_ref[...],
                    0.0)
    out_ref[...] = jnp.dot(h, w_ref[...],
                           preferred_element_type=jnp.float32) * dinv


def _tc_mid(t0, t1, y, dinv, b, w, d_in, d_out):
    return pl.pallas_call(
        _tc_mid_body,
        grid=(_NBLK,),
        in_specs=[
            pl.BlockSpec((_BLK, d_in), lambda i: (i, 0)),
            pl.BlockSpec((_BLK, d_in), lambda i: (i, 0)),
            pl.BlockSpec((_BLK, d_in), lambda i: (i, 0)),
            pl.BlockSpec((_BLK, 1), lambda i: (i, 0)),
            pl.BlockSpec((1, d_in), lambda i: (0, 0)),
            pl.BlockSpec((d_in, d_out), lambda i: (0, 0)),
        ],
        out_specs=pl.BlockSpec((_BLK, d_out), lambda i: (i, 0)),
        out_shape=jax.ShapeDtypeStruct((NPAD, d_out), jnp.float32),
    )(t0, t1, y, dinv, b, w)


def _tc_last_body(t0_ref, t1_ref, y_ref, dinv_ref, b_ref, out_ref):
    out_ref[...] = (dinv_ref[...] * (t0_ref[...] + t1_ref[...] - y_ref[...])
                    + b_ref[...])


def _tc_last(t0, t1, y, dinv, b, d):
    return pl.pallas_call(
        _tc_last_body,
        grid=(_NBLK,),
        in_specs=[
            pl.BlockSpec((_BLK, d), lambda i: (i, 0)),
            pl.BlockSpec((_BLK, d), lambda i: (i, 0)),
            pl.BlockSpec((_BLK, d), lambda i: (i, 0)),
            pl.BlockSpec((_BLK, 1), lambda i: (i, 0)),
            pl.BlockSpec((1, d), lambda i: (0, 0)),
        ],
        out_specs=pl.BlockSpec((_BLK, d), lambda i: (i, 0)),
        out_shape=jax.ShapeDtypeStruct((NPAD, d), jnp.float32),
    )(t0, t1, y, dinv, b)


def kernel(x, edge_index, W1, b1, W2, b2, W3, b3, W4, b4):
    src = edge_index[0].astype(jnp.int32)
    dst = edge_index[1].astype(jnp.int32)
    npad_e = NW * EPW - E
    src3 = jnp.concatenate(
        [src, jnp.zeros((npad_e,), jnp.int32)]).reshape(NW, NCHUNK, CH)
    dst3 = jnp.concatenate(
        [dst, jnp.full((npad_e,), PAD_DST, jnp.int32)]).reshape(NW, NCHUNK, CH)

    x_p = jnp.pad(x, ((0, NPAD - N), (0, 0)))

    deg = _deg_call(dst3)
    d0 = deg[0][:, None]
    d1 = deg[1][:, None]

    y1, dinv = _tc_first(x_p, W1, d0, d1)

    t = _agg64(y1, src3, dst3)
    y2 = _tc_mid(t[0], t[1], y1, dinv, b1[None, :], W2, 64, 64)
    t = _agg64(y2, src3, dst3)
    y3 = _tc_mid(t[0], t[1], y2, dinv, b2[None, :], W3, 64, 64)
    t = _agg64(y3, src3, dst3)
    y4 = _tc_mid(t[0], t[1], y3, dinv, b3[None, :], W4, 64, 32)
    t = _agg32(y4, src3, dst3)
    out = _tc_last(t[0], t[1], y4, dinv, b4[None, :], 32)

    return out[:N]


# speedup vs baseline: 24.6309x; 1.9604x over previous
import functools

import jax
import jax.numpy as jnp
from jax import lax
from jax.experimental import pallas as pl
from jax.experimental.pallas import tpu as pltpu
from jax.experimental.pallas import tpu_sc as plsc

N = 10000
E = 320000
NPAD = 10240
NC, NS = 2, 16
NW = NC * NS
CH = 64
EPW = 10240
NCHUNK = EPW // CH
RPT = NPAD // NS
PAD_DST = NPAD - 8

_MESH = dict(core_axis_name="c", subcore_axis_name="s")


def _deg_body(dst_hbm, deg_hbm, dstv, onesv, stripev, degsp):
    c = lax.axis_index("c")
    s = lax.axis_index("s")
    wid = c * NS + s

    def zero_body(i, _):
        stripev[pl.ds(i * 16, 16)] = jnp.zeros((16,), jnp.float32)
        return 0

    lax.fori_loop(0, RPT // 16, zero_body, 0)

    def ones_body(i, _):
        onesv[pl.ds(i * 16, 16)] = jnp.ones((16,), jnp.float32)
        return 0

    lax.fori_loop(0, CH // 16, ones_body, 0)

    pltpu.sync_copy(stripev, degsp.at[pl.ds(s * RPT, RPT)])
    pltpu.sync_copy(dst_hbm.at[wid], dstv)
    plsc.subcore_barrier()

    def body(i, _):
        pltpu.sync_copy(onesv, degsp.at[dstv.at[i]], add=True)
        return 0

    lax.fori_loop(0, NCHUNK, body, 0)
    plsc.subcore_barrier()

    pltpu.sync_copy(degsp.at[pl.ds(s * RPT, RPT)], stripev)
    pltpu.sync_copy(stripev, deg_hbm.at[c, pl.ds(s * RPT, RPT)])


_SC_PARAMS = pltpu.CompilerParams(use_tc_tiling_on_sc=False)

_deg_call = pl.kernel(
    _deg_body,
    out_type=jax.ShapeDtypeStruct((NC, NPAD), jnp.float32),
    mesh=plsc.VectorSubcoreMesh(**_MESH),
    compiler_params=_SC_PARAMS,
    scratch_types=[
        pltpu.VMEM((NCHUNK, CH), jnp.int32),
        pltpu.VMEM((CH,), jnp.float32),
        pltpu.VMEM((RPT,), jnp.float32),
        pltpu.VMEM_SHARED((NPAD,), jnp.float32),
    ],
)


_NBUF = 4


def _agg_body(y_hbm, src_hbm, dst_hbm, out_hbm, srcv, dstv,
              buf0, buf1, buf2, buf3, ysp, tsp,
              g0, g1, g2, g3, s0, s1, s2, s3):
    c = lax.axis_index("c")
    s = lax.axis_index("s")
    wid = c * NS + s
    bufs = (buf0, buf1, buf2, buf3)
    gsem = (g0, g1, g2, g3)
    ssem = (s0, s1, s2, s3)

    pltpu.sync_copy(y_hbm.at[pl.ds(s * RPT, RPT)], ysp.at[pl.ds(s * RPT, RPT)])
    pltpu.sync_copy(y_hbm.at[pl.ds(s * RPT, RPT)], tsp.at[pl.ds(s * RPT, RPT)])
    pltpu.sync_copy(src_hbm.at[wid], srcv)
    pltpu.sync_copy(dst_hbm.at[wid], dstv)
    plsc.subcore_barrier()

    for j in range(_NBUF):
        pltpu.async_copy(ysp.at[srcv.at[j]], bufs[j], gsem[j])

    def body(i, _):
        scat = []
        for j in range(_NBUF):
            cch = _NBUF * i + j
            pltpu.make_async_copy(ysp.at[srcv.at[cch]], bufs[j],
                                  gsem[j]).wait()
            scat.append(pltpu.async_copy(bufs[j], tsp.at[dstv.at[cch]],
                                         ssem[j], add=True))
        for j in range(_NBUF):
            nch = _NBUF * (i + 1) + j
            scat[j].wait()
            pltpu.async_copy(ysp.at[srcv.at[nch]], bufs[j], gsem[j])
        return 0

    lax.fori_loop(0, NCHUNK // _NBUF - 1, body, 0)
    for j in range(_NBUF):
        cch = NCHUNK - _NBUF + j
        pltpu.make_async_copy(ysp.at[srcv.at[cch]], bufs[j], gsem[j]).wait()
        pltpu.sync_copy(bufs[j], tsp.at[dstv.at[cch]], add=True)
    plsc.subcore_barrier()

    pltpu.sync_copy(tsp.at[pl.ds(s * RPT, RPT)],
                    out_hbm.at[c, pl.ds(s * RPT, RPT)])


def _make_agg(d):
    return pl.kernel(
        _agg_body,
        out_type=jax.ShapeDtypeStruct((NC, NPAD, d), jnp.float32),
        mesh=plsc.VectorSubcoreMesh(**_MESH),
        compiler_params=_SC_PARAMS,
        scratch_types=[
            pltpu.VMEM((NCHUNK, CH), jnp.int32),
            pltpu.VMEM((NCHUNK, CH), jnp.int32),
        ] + [pltpu.VMEM((CH, d), jnp.float32) for _ in range(_NBUF)] + [
            pltpu.VMEM_SHARED((NPAD, d), jnp.float32),
            pltpu.VMEM_SHARED((NPAD, d), jnp.float32),
        ] + [pltpu.SemaphoreType.DMA for _ in range(2 * _NBUF)],
    )


_agg64 = _make_agg(64)
_agg32 = _make_agg(32)


_BLK = 512
_NBLK = NPAD // _BLK


def _tc_first_body(x_ref, w_ref, d0_ref, d1_ref, y_ref, dinv_ref):
    dinv = lax.rsqrt(1.0 + d0_ref[...] + d1_ref[...])
    y_ref[...] = jnp.dot(x_ref[...], w_ref[...],
                         preferred_element_type=jnp.float32) * dinv
    dinv_ref[...] = dinv


def _tc_first(x_p, w1, d0, d1):
    return pl.pallas_call(
        _tc_first_body,
        grid=(_NBLK,),
        in_specs=[
            pl.BlockSpec((_BLK, 128), lambda i: (i, 0)),
            pl.BlockSpec((128, 64), lambda i: (0, 0)),
            pl.BlockSpec((_BLK, 1), lambda i: (i, 0)),
            pl.BlockSpec((_BLK, 1), lambda i: (i, 0)),
        ],
        out_specs=[
            pl.BlockSpec((_BLK, 64), lambda i: (i, 0)),
            pl.BlockSpec((_BLK, 1), lambda i: (i, 0)),
        ],
        out_shape=[
            jax.ShapeDtypeStruct((NPAD, 64), jnp.float32),
            jax.ShapeDtypeStruct((NPAD, 1), jnp.float32),
        ],
    )(x_p, w1, d0, d1)


def _tc_mid_body(t0_ref, t1_ref, y_ref, dinv_ref, b_ref, w_ref, out_ref):
    dinv = dinv_ref[...]
    h = jnp.maximum(dinv * (t0_ref[...] + t1_ref[...] - y_ref[...]) + b_ref[...],
                    0.0)
    out_ref[...] = jnp.dot(h, w_ref[...],
                           preferred_element_type=jnp.float32) * dinv


def _tc_mid(t0, t1, y, dinv, b, w, d_in, d_out):
    return pl.pallas_call(
        _tc_mid_body,
        grid=(_NBLK,),
        in_specs=[
            pl.BlockSpec((_BLK, d_in), lambda i: (i, 0)),
            pl.BlockSpec((_BLK, d_in), lambda i: (i, 0)),
            pl.BlockSpec((_BLK, d_in), lambda i: (i, 0)),
            pl.BlockSpec((_BLK, 1), lambda i: (i, 0)),
            pl.BlockSpec((1, d_in), lambda i: (0, 0)),
            pl.BlockSpec((d_in, d_out), lambda i: (0, 0)),
        ],
        out_specs=pl.BlockSpec((_BLK, d_out), lambda i: (i, 0)),
        out_shape=jax.ShapeDtypeStruct((NPAD, d_out), jnp.float32),
    )(t0, t1, y, dinv, b, w)


def _tc_last_body(t0_ref, t1_ref, y_ref, dinv_ref, b_ref, out_ref):
    out_ref[...] = (dinv_ref[...] * (t0_ref[...] + t1_ref[...] - y_ref[...])
                    + b_ref[...])


def _tc_last(t0, t1, y, dinv, b, d):
    return pl.pallas_call(
        _tc_last_body,
        grid=(_NBLK,),
        in_specs=[
            pl.BlockSpec((_BLK, d), lambda i: (i, 0)),
            pl.BlockSpec((_BLK, d), lambda i: (i, 0)),
            pl.BlockSpec((_BLK, d), lambda i: (i, 0)),
            pl.BlockSpec((_BLK, 1), lambda i: (i, 0)),
            pl.BlockSpec((1, d), lambda i: (0, 0)),
        ],
        out_specs=pl.BlockSpec((_BLK, d), lambda i: (i, 0)),
        out_shape=jax.ShapeDtypeStruct((NPAD, d), jnp.float32),
    )(t0, t1, y, dinv, b)


def kernel(x, edge_index, W1, b1, W2, b2, W3, b3, W4, b4):
    src = edge_index[0].astype(jnp.int32)
    dst = edge_index[1].astype(jnp.int32)
    npad_e = NW * EPW - E
    src3 = jnp.concatenate(
        [src, jnp.zeros((npad_e,), jnp.int32)]).reshape(NW, NCHUNK, CH)
    dst3 = jnp.concatenate(
        [dst, jnp.full((npad_e,), PAD_DST, jnp.int32)]).reshape(NW, NCHUNK, CH)

    x_p = jnp.pad(x, ((0, NPAD - N), (0, 0)))

    deg = _deg_call(dst3)
    d0 = deg[0][:, None]
    d1 = deg[1][:, None]

    y1, dinv = _tc_first(x_p, W1, d0, d1)

    t = _agg64(y1, src3, dst3)
    y2 = _tc_mid(t[0], t[1], y1, dinv, b1[None, :], W2, 64, 64)
    t = _agg64(y2, src3, dst3)
    y3 = _tc_mid(t[0], t[1], y2, dinv, b2[None, :], W3, 64, 64)
    t = _agg64(y3, src3, dst3)
    y4 = _tc_mid(t[0], t[1], y3, dinv, b3[None, :], W4, 64, 32)
    t = _agg32(y4, src3, dst3)
    out = _tc_last(t[0], t[1], y4, dinv, b4[None, :], 32)

    return out[:N]


# speedup vs baseline: 25.2313x; 1.0244x over previous
import functools

import jax
import jax.numpy as jnp
from jax import lax
from jax.experimental import pallas as pl
from jax.experimental.pallas import tpu as pltpu
from jax.experimental.pallas import tpu_sc as plsc

N = 10000
E = 320000
NPAD = 10240
NC, NS = 2, 16
NW = NC * NS
CH = 128
EPW = 10240
NCHUNK = EPW // CH
RPT = NPAD // NS
PAD_DST = NPAD - 8

_MESH = dict(core_axis_name="c", subcore_axis_name="s")


def _deg_body(dst_hbm, deg_hbm, dstv, onesv, stripev, degsp):
    c = lax.axis_index("c")
    s = lax.axis_index("s")
    wid = c * NS + s

    def zero_body(i, _):
        stripev[pl.ds(i * 16, 16)] = jnp.zeros((16,), jnp.float32)
        return 0

    lax.fori_loop(0, RPT // 16, zero_body, 0)

    def ones_body(i, _):
        onesv[pl.ds(i * 16, 16)] = jnp.ones((16,), jnp.float32)
        return 0

    lax.fori_loop(0, CH // 16, ones_body, 0)

    pltpu.sync_copy(stripev, degsp.at[pl.ds(s * RPT, RPT)])
    pltpu.sync_copy(dst_hbm.at[wid], dstv)
    plsc.subcore_barrier()

    def body(i, _):
        pltpu.sync_copy(onesv, degsp.at[dstv.at[i]], add=True)
        return 0

    lax.fori_loop(0, NCHUNK, body, 0)
    plsc.subcore_barrier()

    pltpu.sync_copy(degsp.at[pl.ds(s * RPT, RPT)], stripev)
    pltpu.sync_copy(stripev, deg_hbm.at[c, pl.ds(s * RPT, RPT)])


_SC_PARAMS = pltpu.CompilerParams(use_tc_tiling_on_sc=False)

_deg_call = pl.kernel(
    _deg_body,
    out_type=jax.ShapeDtypeStruct((NC, NPAD), jnp.float32),
    mesh=plsc.VectorSubcoreMesh(**_MESH),
    compiler_params=_SC_PARAMS,
    scratch_types=[
        pltpu.VMEM((NCHUNK, CH), jnp.int32),
        pltpu.VMEM((CH,), jnp.float32),
        pltpu.VMEM((RPT,), jnp.float32),
        pltpu.VMEM_SHARED((NPAD,), jnp.float32),
    ],
)


_NBUF = 3


def _agg_body(y_hbm, src_hbm, dst_hbm, out_hbm, srcv, dstv,
              buf0, buf1, buf2, ysp, tsp,
              g0, g1, g2, s0, s1, s2):
    c = lax.axis_index("c")
    s = lax.axis_index("s")
    wid = c * NS + s
    bufs = (buf0, buf1, buf2)
    gsem = (g0, g1, g2)
    ssem = (s0, s1, s2)

    pltpu.sync_copy(y_hbm.at[pl.ds(s * RPT, RPT)], ysp.at[pl.ds(s * RPT, RPT)])
    pltpu.sync_copy(y_hbm.at[pl.ds(s * RPT, RPT)], tsp.at[pl.ds(s * RPT, RPT)])
    pltpu.sync_copy(src_hbm.at[wid], srcv)
    pltpu.sync_copy(dst_hbm.at[wid], dstv)
    plsc.subcore_barrier()

    for j in range(_NBUF):
        pltpu.async_copy(ysp.at[srcv.at[j]], bufs[j], gsem[j])

    def body(i, _):
        scat = []
        for j in range(_NBUF):
            cch = _NBUF * i + j
            pltpu.make_async_copy(ysp.at[srcv.at[cch]], bufs[j],
                                  gsem[j]).wait()
            scat.append(pltpu.async_copy(bufs[j], tsp.at[dstv.at[cch]],
                                         ssem[j], add=True))
        for j in range(_NBUF):
            nch = _NBUF * (i + 1) + j
            scat[j].wait()
            pltpu.async_copy(ysp.at[srcv.at[nch]], bufs[j], gsem[j])
        return 0

    lax.fori_loop(0, NCHUNK // _NBUF - 1, body, 0)
    last_pref = _NBUF * (NCHUNK // _NBUF)
    for j in range(_NBUF):
        cch = last_pref - _NBUF + j
        pltpu.make_async_copy(ysp.at[srcv.at[cch]], bufs[j], gsem[j]).wait()
        pltpu.sync_copy(bufs[j], tsp.at[dstv.at[cch]], add=True)
    for cch in range(last_pref, NCHUNK):
        pltpu.async_copy(ysp.at[srcv.at[cch]], bufs[0], gsem[0]).wait()
        pltpu.sync_copy(bufs[0], tsp.at[dstv.at[cch]], add=True)
    plsc.subcore_barrier()

    pltpu.sync_copy(tsp.at[pl.ds(s * RPT, RPT)],
                    out_hbm.at[c, pl.ds(s * RPT, RPT)])


def _make_agg(d):
    return pl.kernel(
        _agg_body,
        out_type=jax.ShapeDtypeStruct((NC, NPAD, d), jnp.float32),
        mesh=plsc.VectorSubcoreMesh(**_MESH),
        compiler_params=_SC_PARAMS,
        scratch_types=[
            pltpu.VMEM((NCHUNK, CH), jnp.int32),
            pltpu.VMEM((NCHUNK, CH), jnp.int32),
        ] + [pltpu.VMEM((CH, d), jnp.float32) for _ in range(_NBUF)] + [
            pltpu.VMEM_SHARED((NPAD, d), jnp.float32),
            pltpu.VMEM_SHARED((NPAD, d), jnp.float32),
        ] + [pltpu.SemaphoreType.DMA for _ in range(2 * _NBUF)],
    )


_agg64 = _make_agg(64)
_agg32 = _make_agg(32)


_BLK = 512
_NBLK = NPAD // _BLK


def _tc_first_body(x_ref, w_ref, d0_ref, d1_ref, y_ref, dinv_ref):
    dinv = lax.rsqrt(1.0 + d0_ref[...] + d1_ref[...])
    y_ref[...] = jnp.dot(x_ref[...], w_ref[...],
                         preferred_element_type=jnp.float32) * dinv
    dinv_ref[...] = dinv


def _tc_first(x_p, w1, d0, d1):
    return pl.pallas_call(
        _tc_first_body,
        grid=(_NBLK,),
        in_specs=[
            pl.BlockSpec((_BLK, 128), lambda i: (i, 0)),
            pl.BlockSpec((128, 64), lambda i: (0, 0)),
            pl.BlockSpec((_BLK, 1), lambda i: (i, 0)),
            pl.BlockSpec((_BLK, 1), lambda i: (i, 0)),
        ],
        out_specs=[
            pl.BlockSpec((_BLK, 64), lambda i: (i, 0)),
            pl.BlockSpec((_BLK, 1), lambda i: (i, 0)),
        ],
        out_shape=[
            jax.ShapeDtypeStruct((NPAD, 64), jnp.float32),
            jax.ShapeDtypeStruct((NPAD, 1), jnp.float32),
        ],
    )(x_p, w1, d0, d1)


def _tc_mid_body(t_ref, y_ref, dinv_ref, b_ref, w_ref, out_ref):
    dinv = dinv_ref[...]
    h = jnp.maximum(dinv * (t_ref[0] + t_ref[1] - y_ref[...]) + b_ref[...],
                    0.0)
    out_ref[...] = jnp.dot(h, w_ref[...],
                           preferred_element_type=jnp.float32) * dinv


def _tc_mid(t, y, dinv, b, w, d_in, d_out):
    return pl.pallas_call(
        _tc_mid_body,
        grid=(_NBLK,),
        in_specs=[
            pl.BlockSpec((NC, _BLK, d_in), lambda i: (0, i, 0)),
            pl.BlockSpec((_BLK, d_in), lambda i: (i, 0)),
            pl.BlockSpec((_BLK, 1), lambda i: (i, 0)),
            pl.BlockSpec((1, d_in), lambda i: (0, 0)),
            pl.BlockSpec((d_in, d_out), lambda i: (0, 0)),
        ],
        out_specs=pl.BlockSpec((_BLK, d_out), lambda i: (i, 0)),
        out_shape=jax.ShapeDtypeStruct((NPAD, d_out), jnp.float32),
    )(t, y, dinv, b, w)


def _tc_last_body(t_ref, y_ref, dinv_ref, b_ref, out_ref):
    out_ref[...] = (dinv_ref[...] * (t_ref[0] + t_ref[1] - y_ref[...])
                    + b_ref[...])


def _tc_last(t, y, dinv, b, d):
    return pl.pallas_call(
        _tc_last_body,
        grid=(_NBLK,),
        in_specs=[
            pl.BlockSpec((NC, _BLK, d), lambda i: (0, i, 0)),
            pl.BlockSpec((_BLK, d), lambda i: (i, 0)),
            pl.BlockSpec((_BLK, 1), lambda i: (i, 0)),
            pl.BlockSpec((1, d), lambda i: (0, 0)),
        ],
        out_specs=pl.BlockSpec((_BLK, d), lambda i: (i, 0)),
        out_shape=jax.ShapeDtypeStruct((NPAD, d), jnp.float32),
    )(t, y, dinv, b)


def kernel(x, edge_index, W1, b1, W2, b2, W3, b3, W4, b4):
    src = edge_index[0].astype(jnp.int32)
    dst = edge_index[1].astype(jnp.int32)
    npad_e = NW * EPW - E
    src3 = jnp.concatenate(
        [src, jnp.zeros((npad_e,), jnp.int32)]).reshape(NW, NCHUNK, CH)
    dst3 = jnp.concatenate(
        [dst, jnp.full((npad_e,), PAD_DST, jnp.int32)]).reshape(NW, NCHUNK, CH)

    x_p = jnp.pad(x, ((0, NPAD - N), (0, 0)))

    deg = _deg_call(dst3)
    d0 = deg[0][:, None]
    d1 = deg[1][:, None]

    y1, dinv = _tc_first(x_p, W1, d0, d1)

    t = _agg64(y1, src3, dst3)
    y2 = _tc_mid(t, y1, dinv, b1[None, :], W2, 64, 64)
    t = _agg64(y2, src3, dst3)
    y3 = _tc_mid(t, y2, dinv, b2[None, :], W3, 64, 64)
    t = _agg64(y3, src3, dst3)
    y4 = _tc_mid(t, y3, dinv, b3[None, :], W4, 64, 32)
    t = _agg32(y4, src3, dst3)
    out = _tc_last(t, y4, dinv, b4[None, :], 32)

    return out[:N]


# speedup vs baseline: 25.7401x; 1.0202x over previous
import functools

import jax
import jax.numpy as jnp
from jax import lax
from jax.experimental import pallas as pl
from jax.experimental.pallas import tpu as pltpu
from jax.experimental.pallas import tpu_sc as plsc

N = 10000
E = 320000
NPAD = 10240
NC, NS = 2, 16
NW = NC * NS
CH = 64
EPW = 10240
NCHUNK = EPW // CH
RPT = NPAD // NS
PAD_DST = NPAD - 8

_MESH = dict(core_axis_name="c", subcore_axis_name="s")


def _deg_body(dst_hbm, deg_hbm, dstv, onesv, stripev, degsp):
    c = lax.axis_index("c")
    s = lax.axis_index("s")
    wid = c * NS + s

    def zero_body(i, _):
        stripev[pl.ds(i * 16, 16)] = jnp.zeros((16,), jnp.float32)
        return 0

    lax.fori_loop(0, RPT // 16, zero_body, 0)

    def ones_body(i, _):
        onesv[pl.ds(i * 16, 16)] = jnp.ones((16,), jnp.float32)
        return 0

    lax.fori_loop(0, CH // 16, ones_body, 0)

    pltpu.sync_copy(stripev, degsp.at[pl.ds(s * RPT, RPT)])
    pltpu.sync_copy(dst_hbm.at[wid], dstv)
    plsc.subcore_barrier()

    def body(i, _):
        pltpu.sync_copy(onesv, degsp.at[dstv.at[i]], add=True)
        return 0

    lax.fori_loop(0, NCHUNK, body, 0)
    plsc.subcore_barrier()

    pltpu.sync_copy(degsp.at[pl.ds(s * RPT, RPT)], stripev)
    pltpu.sync_copy(stripev, deg_hbm.at[c, pl.ds(s * RPT, RPT)])


_SC_PARAMS = pltpu.CompilerParams(use_tc_tiling_on_sc=False)

_deg_call = pl.kernel(
    _deg_body,
    out_type=jax.ShapeDtypeStruct((NC, NPAD), jnp.float32),
    mesh=plsc.VectorSubcoreMesh(**_MESH),
    compiler_params=_SC_PARAMS,
    scratch_types=[
        pltpu.VMEM((NCHUNK, CH), jnp.int32),
        pltpu.VMEM((CH,), jnp.float32),
        pltpu.VMEM((RPT,), jnp.float32),
        pltpu.VMEM_SHARED((NPAD,), jnp.float32),
    ],
)


_NBUF = 3


def _agg_body(y_hbm, src_hbm, dst_hbm, out_hbm, srcv, dstv,
              buf0, buf1, buf2, ysp, tsp,
              g0, g1, g2, s0, s1, s2):
    c = lax.axis_index("c")
    s = lax.axis_index("s")
    wid = c * NS + s
    bufs = (buf0, buf1, buf2)
    gsem = (g0, g1, g2)
    ssem = (s0, s1, s2)

    pltpu.sync_copy(y_hbm.at[pl.ds(s * RPT, RPT)], ysp.at[pl.ds(s * RPT, RPT)])
    pltpu.sync_copy(y_hbm.at[pl.ds(s * RPT, RPT)], tsp.at[pl.ds(s * RPT, RPT)])
    pltpu.sync_copy(src_hbm.at[wid], srcv)
    pltpu.sync_copy(dst_hbm.at[wid], dstv)
    plsc.subcore_barrier()

    for j in range(_NBUF):
        pltpu.async_copy(ysp.at[srcv.at[j]], bufs[j], gsem[j])

    def body(i, _):
        scat = []
        for j in range(_NBUF):
            cch = _NBUF * i + j
            pltpu.make_async_copy(ysp.at[srcv.at[cch]], bufs[j],
                                  gsem[j]).wait()
            scat.append(pltpu.async_copy(bufs[j], tsp.at[dstv.at[cch]],
                                         ssem[j], add=True))
        for j in range(_NBUF):
            nch = _NBUF * (i + 1) + j
            scat[j].wait()
            pltpu.async_copy(ysp.at[srcv.at[nch]], bufs[j], gsem[j])
        return 0

    lax.fori_loop(0, NCHUNK // _NBUF - 1, body, 0)
    last_pref = _NBUF * (NCHUNK // _NBUF)
    for j in range(_NBUF):
        cch = last_pref - _NBUF + j
        pltpu.make_async_copy(ysp.at[srcv.at[cch]], bufs[j], gsem[j]).wait()
        pltpu.sync_copy(bufs[j], tsp.at[dstv.at[cch]], add=True)
    for cch in range(last_pref, NCHUNK):
        pltpu.async_copy(ysp.at[srcv.at[cch]], bufs[0], gsem[0]).wait()
        pltpu.sync_copy(bufs[0], tsp.at[dstv.at[cch]], add=True)
    plsc.subcore_barrier()

    pltpu.sync_copy(tsp.at[pl.ds(s * RPT, RPT)],
                    out_hbm.at[c, pl.ds(s * RPT, RPT)])


def _make_agg(d):
    return pl.kernel(
        _agg_body,
        out_type=jax.ShapeDtypeStruct((NC, NPAD, d), jnp.float32),
        mesh=plsc.VectorSubcoreMesh(**_MESH),
        compiler_params=_SC_PARAMS,
        scratch_types=[
            pltpu.VMEM((NCHUNK, CH), jnp.int32),
            pltpu.VMEM((NCHUNK, CH), jnp.int32),
        ] + [pltpu.VMEM((CH, d), jnp.float32) for _ in range(_NBUF)] + [
            pltpu.VMEM_SHARED((NPAD, d), jnp.float32),
            pltpu.VMEM_SHARED((NPAD, d), jnp.float32),
        ] + [pltpu.SemaphoreType.DMA for _ in range(2 * _NBUF)],
    )


_agg64 = _make_agg(64)
_agg32 = _make_agg(32)


_BLK = 512
_NBLK = NPAD // _BLK


def _tc_first_body(x_ref, w_ref, d0_ref, d1_ref, y_ref, dinv_ref):
    dinv = lax.rsqrt(1.0 + d0_ref[...] + d1_ref[...])
    y_ref[...] = jnp.dot(x_ref[...], w_ref[...],
                         preferred_element_type=jnp.float32) * dinv
    dinv_ref[...] = dinv


def _tc_first(x_p, w1, d0, d1):
    return pl.pallas_call(
        _tc_first_body,
        grid=(_NBLK,),
        in_specs=[
            pl.BlockSpec((_BLK, 128), lambda i: (i, 0)),
            pl.BlockSpec((128, 64), lambda i: (0, 0)),
            pl.BlockSpec((_BLK, 1), lambda i: (i, 0)),
            pl.BlockSpec((_BLK, 1), lambda i: (i, 0)),
        ],
        out_specs=[
            pl.BlockSpec((_BLK, 64), lambda i: (i, 0)),
            pl.BlockSpec((_BLK, 1), lambda i: (i, 0)),
        ],
        out_shape=[
            jax.ShapeDtypeStruct((NPAD, 64), jnp.float32),
            jax.ShapeDtypeStruct((NPAD, 1), jnp.float32),
        ],
    )(x_p, w1, d0, d1)


def _tc_mid_body(t_ref, y_ref, dinv_ref, b_ref, w_ref, out_ref):
    dinv = dinv_ref[...]
    h = jnp.maximum(dinv * (t_ref[0] + t_ref[1] - y_ref[...]) + b_ref[...],
                    0.0)
    out_ref[...] = jnp.dot(h, w_ref[...],
                           preferred_element_type=jnp.float32) * dinv


def _tc_mid(t, y, dinv, b, w, d_in, d_out):
    return pl.pallas_call(
        _tc_mid_body,
        grid=(_NBLK,),
        in_specs=[
            pl.BlockSpec((NC, _BLK, d_in), lambda i: (0, i, 0)),
            pl.BlockSpec((_BLK, d_in), lambda i: (i, 0)),
            pl.BlockSpec((_BLK, 1), lambda i: (i, 0)),
            pl.BlockSpec((1, d_in), lambda i: (0, 0)),
            pl.BlockSpec((d_in, d_out), lambda i: (0, 0)),
        ],
        out_specs=pl.BlockSpec((_BLK, d_out), lambda i: (i, 0)),
        out_shape=jax.ShapeDtypeStruct((NPAD, d_out), jnp.float32),
    )(t, y, dinv, b, w)


def _tc_last_body(t_ref, y_ref, dinv_ref, b_ref, out_ref):
    out_ref[...] = (dinv_ref[...] * (t_ref[0] + t_ref[1] - y_ref[...])
                    + b_ref[...])


def _tc_last(t, y, dinv, b, d):
    return pl.pallas_call(
        _tc_last_body,
        grid=(_NBLK,),
        in_specs=[
            pl.BlockSpec((NC, _BLK, d), lambda i: (0, i, 0)),
            pl.BlockSpec((_BLK, d), lambda i: (i, 0)),
            pl.BlockSpec((_BLK, 1), lambda i: (i, 0)),
            pl.BlockSpec((1, d), lambda i: (0, 0)),
        ],
        out_specs=pl.BlockSpec((_BLK, d), lambda i: (i, 0)),
        out_shape=jax.ShapeDtypeStruct((NPAD, d), jnp.float32),
    )(t, y, dinv, b)


def kernel(x, edge_index, W1, b1, W2, b2, W3, b3, W4, b4):
    src = edge_index[0].astype(jnp.int32)
    dst = edge_index[1].astype(jnp.int32)
    npad_e = NW * EPW - E
    src3 = jnp.concatenate(
        [src, jnp.zeros((npad_e,), jnp.int32)]).reshape(NW, NCHUNK, CH)
    dst3 = jnp.concatenate(
        [dst, jnp.full((npad_e,), PAD_DST, jnp.int32)]).reshape(NW, NCHUNK, CH)

    x_p = jnp.pad(x, ((0, NPAD - N), (0, 0)))

    deg = _deg_call(dst3)
    d0 = deg[0][:, None]
    d1 = deg[1][:, None]

    y1, dinv = _tc_first(x_p, W1, d0, d1)

    t = _agg64(y1, src3, dst3)
    y2 = _tc_mid(t, y1, dinv, b1[None, :], W2, 64, 64)
    t = _agg64(y2, src3, dst3)
    y3 = _tc_mid(t, y2, dinv, b2[None, :], W3, 64, 64)
    t = _agg64(y3, src3, dst3)
    y4 = _tc_mid(t, y3, dinv, b3[None, :], W4, 64, 32)
    t = _agg32(y4, src3, dst3)
    out = _tc_last(t, y4, dinv, b4[None, :], 32)

    return out[:N]


# speedup vs baseline: 28.3123x; 1.0999x over previous
import functools

import jax
import jax.numpy as jnp
from jax import lax
from jax.experimental import pallas as pl
from jax.experimental.pallas import tpu as pltpu
from jax.experimental.pallas import tpu_sc as plsc

N = 10000
E = 320000
NPAD = 10240
NH = NPAD // 2
NC, NS = 2, 16
NW = NC * NS
CH = 64
EPW = 10240
NCHUNK = EPW // CH
RPT = NPAD // NS
NPT = NPAD // NW
PAD_DST = NPAD - 8

_MESH = dict(core_axis_name="c", subcore_axis_name="s")
_SC_PARAMS = pltpu.CompilerParams(use_tc_tiling_on_sc=False,
                                  needs_layout_passes=False)


def _deg_body(dst_hbm, degw_hbm, dstv, onesv, zerov, degbuf, widev, degsp):
    c = lax.axis_index("c")
    s = lax.axis_index("s")

    def zero_body(i, _):
        zerov[pl.ds(i * 16, 16)] = jnp.zeros((16,), jnp.float32)
        return 0

    lax.fori_loop(0, RPT // 16, zero_body, 0)

    def ones_body(i, _):
        onesv[pl.ds(i * 16, 16)] = jnp.ones((16,), jnp.float32)
        return 0

    lax.fori_loop(0, CH // 16, ones_body, 0)

    pltpu.sync_copy(zerov, degsp.at[pl.ds(s * RPT, RPT)])
    plsc.subcore_barrier()

    for k in range(2):
        pltpu.sync_copy(dst_hbm.at[2 * s + k], dstv)

        def body(i, _):
            pltpu.sync_copy(onesv, degsp.at[dstv.at[i]], add=True)
            return 0

        lax.fori_loop(0, NCHUNK, body, 0)
    plsc.subcore_barrier()

    wid = c * NS + s
    n0 = wid * NPT
    pltpu.sync_copy(degsp.at[pl.ds(n0, NPT)], degbuf)

    def wide_body(n, _):
        splat = plsc.load_gather(degbuf, [jnp.full((16,), n, jnp.int32)])
        row = n >> 1
        colb = (n & 1) * 64
        for r in range(4):
            widev[row, pl.ds(colb + r * 16, 16)] = splat
        return 0

    lax.fori_loop(0, NPT, wide_body, 0)
    pltpu.sync_copy(widev, degw_hbm.at[pl.ds(wid * (NPT // 2), NPT // 2)])


_deg_call = pl.kernel(
    _deg_body,
    out_type=jax.ShapeDtypeStruct((NH, 128), jnp.float32),
    mesh=plsc.VectorSubcoreMesh(**_MESH),
    compiler_params=_SC_PARAMS,
    scratch_types=[
        pltpu.VMEM((NCHUNK, CH), jnp.int32),
        pltpu.VMEM((CH,), jnp.float32),
        pltpu.VMEM((RPT,), jnp.float32),
        pltpu.VMEM((NPT,), jnp.float32),
        pltpu.VMEM((NPT // 2, 128), jnp.float32),
        pltpu.VMEM_SHARED((NPAD,), jnp.float32),
    ],
)


_NBUF = 3


def _agg_body(y_hbm, src_hbm, dst_hbm, out_hbm, srcv, dstv,
              buf0, buf1, buf2, ysp, tsp,
              g0, g1, g2, s0, s1, s2):
    c = lax.axis_index("c")
    s = lax.axis_index("s")
    wid = c * NS + s
    bufs = (buf0, buf1, buf2)
    gsem = (g0, g1, g2)
    ssem = (s0, s1, s2)

    pltpu.sync_copy(y_hbm.at[pl.ds(s * RPT, RPT)], ysp.at[pl.ds(s * RPT, RPT)])
    pltpu.sync_copy(y_hbm.at[pl.ds(s * RPT, RPT)], tsp.at[pl.ds(s * RPT, RPT)])
    pltpu.sync_copy(src_hbm.at[wid], srcv)
    pltpu.sync_copy(dst_hbm.at[wid], dstv)
    plsc.subcore_barrier()

    for j in range(_NBUF):
        pltpu.async_copy(ysp.at[srcv.at[j]], bufs[j], gsem[j])

    def body(i, _):
        scat = []
        for j in range(_NBUF):
            cch = _NBUF * i + j
            pltpu.make_async_copy(ysp.at[srcv.at[cch]], bufs[j],
                                  gsem[j]).wait()
            scat.append(pltpu.async_copy(bufs[j], tsp.at[dstv.at[cch]],
                                         ssem[j], add=True))
        for j in range(_NBUF):
            nch = _NBUF * (i + 1) + j
            scat[j].wait()
            pltpu.async_copy(ysp.at[srcv.at[nch]], bufs[j], gsem[j])
        return 0

    lax.fori_loop(0, NCHUNK // _NBUF - 1, body, 0)
    last_pref = _NBUF * (NCHUNK // _NBUF)
    for j in range(_NBUF):
        cch = last_pref - _NBUF + j
        pltpu.make_async_copy(ysp.at[srcv.at[cch]], bufs[j], gsem[j]).wait()
        pltpu.sync_copy(bufs[j], tsp.at[dstv.at[cch]], add=True)
    for cch in range(last_pref, NCHUNK):
        pltpu.async_copy(ysp.at[srcv.at[cch]], bufs[0], gsem[0]).wait()
        pltpu.sync_copy(bufs[0], tsp.at[dstv.at[cch]], add=True)
    plsc.subcore_barrier()

    pltpu.sync_copy(tsp.at[pl.ds(s * RPT, RPT)],
                    out_hbm.at[c, pl.ds(s * RPT, RPT)])


def _make_agg(d):
    return pl.kernel(
        _agg_body,
        out_type=jax.ShapeDtypeStruct((NC, NPAD, d), jnp.float32),
        mesh=plsc.VectorSubcoreMesh(**_MESH),
        compiler_params=_SC_PARAMS,
        scratch_types=[
            pltpu.VMEM((NCHUNK, CH), jnp.int32),
            pltpu.VMEM((NCHUNK, CH), jnp.int32),
        ] + [pltpu.VMEM((CH, d), jnp.float32) for _ in range(_NBUF)] + [
            pltpu.VMEM_SHARED((NPAD, d), jnp.float32),
            pltpu.VMEM_SHARED((NPAD, d), jnp.float32),
        ] + [pltpu.SemaphoreType.DMA for _ in range(2 * _NBUF)],
    )


_agg64 = _make_agg(64)
_agg32 = _make_agg(32)


_BLK = 256
_NBLK = NH // _BLK


def _tc_first_body(x_ref, w_ref, degw_ref, y_ref, dinv_ref):
    dinv = lax.rsqrt(1.0 + degw_ref[...])
    y_ref[...] = jnp.dot(x_ref[...], w_ref[...],
                         preferred_element_type=jnp.float32) * dinv
    dinv_ref[...] = dinv


def _tc_first(x_pair, w1_2, degw):
    return pl.pallas_call(
        _tc_first_body,
        grid=(_NBLK,),
        in_specs=[
            pl.BlockSpec((_BLK, 256), lambda i: (i, 0)),
            pl.BlockSpec((256, 128), lambda i: (0, 0)),
            pl.BlockSpec((_BLK, 128), lambda i: (i, 0)),
        ],
        out_specs=[
            pl.BlockSpec((_BLK, 128), lambda i: (i, 0)),
            pl.BlockSpec((_BLK, 128), lambda i: (i, 0)),
        ],
        out_shape=[
            jax.ShapeDtypeStruct((NH, 128), jnp.float32),
            jax.ShapeDtypeStruct((NH, 128), jnp.float32),
        ],
    )(x_pair, w1_2, degw)


def _dinv32(dv):
    return jnp.concatenate([dv[:, 0:32], dv[:, 64:96]], axis=1)


def _tc_mid_body(t_ref, y_ref, dinv_ref, b_ref, w_ref, out_ref, *, d_out):
    dinv = dinv_ref[...]
    h = jnp.maximum(dinv * (t_ref[0] + t_ref[1] - y_ref[...]) + b_ref[...],
                    0.0)
    scale = dinv if d_out == 128 else _dinv32(dinv)
    out_ref[...] = jnp.dot(h, w_ref[...],
                           preferred_element_type=jnp.float32) * scale


def _tc_mid(t, y, dinv, b, w, d_out):
    return pl.pallas_call(
        functools.partial(_tc_mid_body, d_out=d_out),
        grid=(_NBLK,),
        in_specs=[
            pl.BlockSpec((NC, _BLK, 128), lambda i: (0, i, 0)),
            pl.BlockSpec((_BLK, 128), lambda i: (i, 0)),
            pl.BlockSpec((_BLK, 128), lambda i: (i, 0)),
            pl.BlockSpec((1, 128), lambda i: (0, 0)),
            pl.BlockSpec((128, d_out), lambda i: (0, 0)),
        ],
        out_specs=pl.BlockSpec((_BLK, d_out), lambda i: (i, 0)),
        out_shape=jax.ShapeDtypeStruct((NH, d_out), jnp.float32),
    )(t, y, dinv, b, w)


def _tc_last_body(t_ref, y_ref, dinv_ref, b_ref, out_ref):
    out_ref[...] = (_dinv32(dinv_ref[...]) * (t_ref[0] + t_ref[1] - y_ref[...])
                    + b_ref[...])


def _tc_last(t, y, dinv, b):
    return pl.pallas_call(
        _tc_last_body,
        grid=(_NBLK,),
        in_specs=[
            pl.BlockSpec((NC, _BLK, 64), lambda i: (0, i, 0)),
            pl.BlockSpec((_BLK, 64), lambda i: (i, 0)),
            pl.BlockSpec((_BLK, 128), lambda i: (i, 0)),
            pl.BlockSpec((1, 64), lambda i: (0, 0)),
        ],
        out_specs=pl.BlockSpec((_BLK, 64), lambda i: (i, 0)),
        out_shape=jax.ShapeDtypeStruct((NH, 64), jnp.float32),
    )(t, y, dinv, b)


def _blockdiag2(w):
    d_in, d_out = w.shape
    z = jnp.zeros((2 * d_in, 2 * d_out), jnp.float32)
    return z.at[:d_in, :d_out].set(w).at[d_in:, d_out:].set(w)


def kernel(x, edge_index, W1, b1, W2, b2, W3, b3, W4, b4):
    src = edge_index[0].astype(jnp.int32)
    dst = edge_index[1].astype(jnp.int32)
    npad_e = NW * EPW - E
    src3 = jnp.concatenate(
        [src, jnp.zeros((npad_e,), jnp.int32)]).reshape(NW, NCHUNK, CH)
    dst3 = jnp.concatenate(
        [dst, jnp.full((npad_e,), PAD_DST, jnp.int32)]).reshape(NW, NCHUNK, CH)

    x_pair = jnp.pad(x, ((0, NPAD - N), (0, 0))).reshape(NH, 256)
    w1_2 = _blockdiag2(W1)
    w2_2 = _blockdiag2(W2)
    w3_2 = _blockdiag2(W3)
    w4_2 = _blockdiag2(W4)
    b1p = jnp.concatenate([b1, b1])[None, :]
    b2p = jnp.concatenate([b2, b2])[None, :]
    b3p = jnp.concatenate([b3, b3])[None, :]
    b4p = jnp.concatenate([b4, b4])[None, :]

    degw = _deg_call(dst3)
    y1, dinvw = _tc_first(x_pair, w1_2, degw)

    t = _agg64(y1.reshape(NPAD, 64), src3, dst3)
    y2 = _tc_mid(t.reshape(NC, NH, 128), y1, dinvw, b1p, w2_2, 128)
    t = _agg64(y2.reshape(NPAD, 64), src3, dst3)
    y3 = _tc_mid(t.reshape(NC, NH, 128), y2, dinvw, b2p, w3_2, 128)
    t = _agg64(y3.reshape(NPAD, 64), src3, dst3)
    y4 = _tc_mid(t.reshape(NC, NH, 128), y3, dinvw, b3p, w4_2, 64)
    t = _agg32(y4.reshape(NPAD, 32), src3, dst3)
    out = _tc_last(t.reshape(NC, NH, 64), y4, dinvw, b4p)

    return out.reshape(NPAD, 32)[:N]


# speedup vs baseline: 30.2242x; 1.0675x over previous
import functools

import jax
import jax.numpy as jnp
from jax import lax
from jax.experimental import pallas as pl
from jax.experimental.pallas import tpu as pltpu
from jax.experimental.pallas import tpu_sc as plsc

N = 10000
E = 320000
NPAD = 10240
NH = NPAD // 2
NC, NS = 2, 16
NW = NC * NS
CH = 80
EPW = E // NW
NCHUNK = EPW // CH
RPT = NPAD // NS
NPT = NPAD // NW

_MESH = dict(core_axis_name="c", subcore_axis_name="s")
_SC_PARAMS = pltpu.CompilerParams(use_tc_tiling_on_sc=False,
                                  needs_layout_passes=False)


def _deg_body(dst_hbm, degw_hbm, dstv, onesv, zerov, degbuf, widev, degsp):
    c = lax.axis_index("c")
    s = lax.axis_index("s")

    def zero_body(i, _):
        zerov[pl.ds(i * 16, 16)] = jnp.zeros((16,), jnp.float32)
        return 0

    lax.fori_loop(0, RPT // 16, zero_body, 0)

    def ones_body(i, _):
        onesv[pl.ds(i * 16, 16)] = jnp.ones((16,), jnp.float32)
        return 0

    lax.fori_loop(0, CH // 16, ones_body, 0)

    pltpu.sync_copy(zerov, degsp.at[pl.ds(s * RPT, RPT)])
    plsc.subcore_barrier()

    for k in range(2):
        pltpu.sync_copy(dst_hbm.at[2 * s + k], dstv)

        def body(i, _):
            pltpu.sync_copy(onesv, degsp.at[dstv.at[i]], add=True)
            return 0

        lax.fori_loop(0, NCHUNK, body, 0)
    plsc.subcore_barrier()

    wid = c * NS + s
    n0 = wid * NPT
    pltpu.sync_copy(degsp.at[pl.ds(n0, NPT)], degbuf)

    def wide_body(n, _):
        splat = plsc.load_gather(degbuf, [jnp.full((16,), n, jnp.int32)])
        row = n >> 1
        colb = (n & 1) * 64
        for r in range(4):
            widev[row, pl.ds(colb + r * 16, 16)] = splat
        return 0

    lax.fori_loop(0, NPT, wide_body, 0)
    pltpu.sync_copy(widev, degw_hbm.at[pl.ds(wid * (NPT // 2), NPT // 2)])


_deg_call = pl.kernel(
    _deg_body,
    out_type=jax.ShapeDtypeStruct((NH, 128), jnp.float32),
    mesh=plsc.VectorSubcoreMesh(**_MESH),
    compiler_params=_SC_PARAMS,
    scratch_types=[
        pltpu.VMEM((NCHUNK, CH), jnp.int32),
        pltpu.VMEM((CH,), jnp.float32),
        pltpu.VMEM((RPT,), jnp.float32),
        pltpu.VMEM((NPT,), jnp.float32),
        pltpu.VMEM((NPT // 2, 128), jnp.float32),
        pltpu.VMEM_SHARED((NPAD,), jnp.float32),
    ],
)


_NBUF = 4


def _agg_body(y_hbm, src_hbm, dst_hbm, out_hbm, srcv, dstv,
              buf0, buf1, buf2, buf3, ysp, tsp,
              g0, g1, g2, g3, s0, s1, s2, s3):
    c = lax.axis_index("c")
    s = lax.axis_index("s")
    wid = c * NS + s
    bufs = (buf0, buf1, buf2, buf3)
    gsem = (g0, g1, g2, g3)
    ssem = (s0, s1, s2, s3)

    pltpu.sync_copy(y_hbm.at[pl.ds(s * RPT, RPT)], ysp.at[pl.ds(s * RPT, RPT)])
    pltpu.sync_copy(y_hbm.at[pl.ds(s * RPT, RPT)], tsp.at[pl.ds(s * RPT, RPT)])
    pltpu.sync_copy(src_hbm.at[wid], srcv)
    pltpu.sync_copy(dst_hbm.at[wid], dstv)
    plsc.subcore_barrier()

    for j in range(_NBUF):
        pltpu.async_copy(ysp.at[srcv.at[j]], bufs[j], gsem[j])

    def body(i, _):
        scat = []
        for j in range(_NBUF):
            cch = _NBUF * i + j
            pltpu.make_async_copy(ysp.at[srcv.at[cch]], bufs[j],
                                  gsem[j]).wait()
            scat.append(pltpu.async_copy(bufs[j], tsp.at[dstv.at[cch]],
                                         ssem[j], add=True))
        for j in range(_NBUF):
            nch = _NBUF * (i + 1) + j
            scat[j].wait()
            pltpu.async_copy(ysp.at[srcv.at[nch]], bufs[j], gsem[j])
        return 0

    lax.fori_loop(0, NCHUNK // _NBUF - 1, body, 0)
    last_pref = _NBUF * (NCHUNK // _NBUF)
    for j in range(_NBUF):
        cch = last_pref - _NBUF + j
        pltpu.make_async_copy(ysp.at[srcv.at[cch]], bufs[j], gsem[j]).wait()
        pltpu.sync_copy(bufs[j], tsp.at[dstv.at[cch]], add=True)
    for cch in range(last_pref, NCHUNK):
        pltpu.async_copy(ysp.at[srcv.at[cch]], bufs[0], gsem[0]).wait()
        pltpu.sync_copy(bufs[0], tsp.at[dstv.at[cch]], add=True)
    plsc.subcore_barrier()

    pltpu.sync_copy(tsp.at[pl.ds(s * RPT, RPT)],
                    out_hbm.at[c, pl.ds(s * RPT, RPT)])


def _make_agg(d):
    return pl.kernel(
        _agg_body,
        out_type=jax.ShapeDtypeStruct((NC, NPAD, d), jnp.float32),
        mesh=plsc.VectorSubcoreMesh(**_MESH),
        compiler_params=_SC_PARAMS,
        scratch_types=[
            pltpu.VMEM((NCHUNK, CH), jnp.int32),
            pltpu.VMEM((NCHUNK, CH), jnp.int32),
        ] + [pltpu.VMEM((CH, d), jnp.float32) for _ in range(_NBUF)] + [
            pltpu.VMEM_SHARED((NPAD, d), jnp.float32),
            pltpu.VMEM_SHARED((NPAD, d), jnp.float32),
        ] + [pltpu.SemaphoreType.DMA for _ in range(2 * _NBUF)],
    )


_agg64 = _make_agg(64)
_agg32 = _make_agg(32)


_BLK = 256
_NBLK = NH // _BLK


def _tc_first_body(x_ref, w_ref, degw_ref, y_ref, dinv_ref):
    dinv = lax.rsqrt(1.0 + degw_ref[...])
    y_ref[...] = jnp.dot(x_ref[...], w_ref[...],
                         preferred_element_type=jnp.float32) * dinv
    dinv_ref[...] = dinv


def _tc_first(x_pair, w1_2, degw):
    return pl.pallas_call(
        _tc_first_body,
        grid=(_NBLK,),
        in_specs=[
            pl.BlockSpec((_BLK, 256), lambda i: (i, 0)),
            pl.BlockSpec((256, 128), lambda i: (0, 0)),
            pl.BlockSpec((_BLK, 128), lambda i: (i, 0)),
        ],
        out_specs=[
            pl.BlockSpec((_BLK, 128), lambda i: (i, 0)),
            pl.BlockSpec((_BLK, 128), lambda i: (i, 0)),
        ],
        out_shape=[
            jax.ShapeDtypeStruct((NH, 128), jnp.float32),
            jax.ShapeDtypeStruct((NH, 128), jnp.float32),
        ],
    )(x_pair, w1_2, degw)


def _dinv32(dv):
    return jnp.concatenate([dv[:, 0:32], dv[:, 64:96]], axis=1)


def _tc_mid_body(t_ref, y_ref, dinv_ref, b_ref, w_ref, out_ref, *, d_out):
    dinv = dinv_ref[...]
    h = jnp.maximum(dinv * (t_ref[0] + t_ref[1] - y_ref[...]) + b_ref[...],
                    0.0)
    scale = dinv if d_out == 128 else _dinv32(dinv)
    out_ref[...] = jnp.dot(h, w_ref[...],
                           preferred_element_type=jnp.float32) * scale


def _tc_mid(t, y, dinv, b, w, d_out):
    return pl.pallas_call(
        functools.partial(_tc_mid_body, d_out=d_out),
        grid=(_NBLK,),
        in_specs=[
            pl.BlockSpec((NC, _BLK, 128), lambda i: (0, i, 0)),
            pl.BlockSpec((_BLK, 128), lambda i: (i, 0)),
            pl.BlockSpec((_BLK, 128), lambda i: (i, 0)),
            pl.BlockSpec((1, 128), lambda i: (0, 0)),
            pl.BlockSpec((128, d_out), lambda i: (0, 0)),
        ],
        out_specs=pl.BlockSpec((_BLK, d_out), lambda i: (i, 0)),
        out_shape=jax.ShapeDtypeStruct((NH, d_out), jnp.float32),
    )(t, y, dinv, b, w)


def _tc_last_body(t_ref, y_ref, dinv_ref, b_ref, out_ref):
    out_ref[...] = (_dinv32(dinv_ref[...]) * (t_ref[0] + t_ref[1] - y_ref[...])
                    + b_ref[...])


def _tc_last(t, y, dinv, b):
    return pl.pallas_call(
        _tc_last_body,
        grid=(_NBLK,),
        in_specs=[
            pl.BlockSpec((NC, _BLK, 64), lambda i: (0, i, 0)),
            pl.BlockSpec((_BLK, 64), lambda i: (i, 0)),
            pl.BlockSpec((_BLK, 128), lambda i: (i, 0)),
            pl.BlockSpec((1, 64), lambda i: (0, 0)),
        ],
        out_specs=pl.BlockSpec((_BLK, 64), lambda i: (i, 0)),
        out_shape=jax.ShapeDtypeStruct((NH, 64), jnp.float32),
    )(t, y, dinv, b)


def _blockdiag2(w):
    d_in, d_out = w.shape
    z = jnp.zeros((2 * d_in, 2 * d_out), jnp.float32)
    return z.at[:d_in, :d_out].set(w).at[d_in:, d_out:].set(w)


def kernel(x, edge_index, W1, b1, W2, b2, W3, b3, W4, b4):
    src = edge_index[0].astype(jnp.int32)
    dst = edge_index[1].astype(jnp.int32)
    src3 = src.reshape(NW, NCHUNK, CH)
    dst3 = dst.reshape(NW, NCHUNK, CH)

    x_pair = jnp.pad(x, ((0, NPAD - N), (0, 0))).reshape(NH, 256)
    w1_2 = _blockdiag2(W1)
    w2_2 = _blockdiag2(W2)
    w3_2 = _blockdiag2(W3)
    w4_2 = _blockdiag2(W4)
    b1p = jnp.concatenate([b1, b1])[None, :]
    b2p = jnp.concatenate([b2, b2])[None, :]
    b3p = jnp.concatenate([b3, b3])[None, :]
    b4p = jnp.concatenate([b4, b4])[None, :]

    degw = _deg_call(dst3)
    y1, dinvw = _tc_first(x_pair, w1_2, degw)

    t = _agg64(y1.reshape(NPAD, 64), src3, dst3)
    y2 = _tc_mid(t.reshape(NC, NH, 128), y1, dinvw, b1p, w2_2, 128)
    t = _agg64(y2.reshape(NPAD, 64), src3, dst3)
    y3 = _tc_mid(t.reshape(NC, NH, 128), y2, dinvw, b2p, w3_2, 128)
    t = _agg64(y3.reshape(NPAD, 64), src3, dst3)
    y4 = _tc_mid(t.reshape(NC, NH, 128), y3, dinvw, b3p, w4_2, 64)
    t = _agg32(y4.reshape(NPAD, 32), src3, dst3)
    out = _tc_last(t.reshape(NC, NH, 64), y4, dinvw, b4p)

    return out.reshape(NPAD, 32)[:N]


# speedup vs baseline: 32.6212x; 1.0793x over previous
import functools

import jax
import jax.numpy as jnp
from jax import lax
from jax.experimental import pallas as pl
from jax.experimental.pallas import tpu as pltpu
from jax.experimental.pallas import tpu_sc as plsc

N = 10000
E = 320000
NPAD = 10240
NH = NPAD // 2
NC, NS = 2, 16
NW = NC * NS
CH = 80
EPW = E // NW
NCHUNK = EPW // CH
RPT = NPAD // NS
NPT = NPAD // NW

_MESH = dict(core_axis_name="c", subcore_axis_name="s")
_SC_PARAMS = pltpu.CompilerParams(use_tc_tiling_on_sc=False,
                                  needs_layout_passes=False)


def _deg_body(dst_hbm, degw_hbm, dstv, onesv, zerov, degbuf, widev, degsp):
    c = lax.axis_index("c")
    s = lax.axis_index("s")

    def zero_body(i, _):
        zerov[pl.ds(i * 16, 16)] = jnp.zeros((16,), jnp.float32)
        return 0

    lax.fori_loop(0, RPT // 16, zero_body, 0)

    def ones_body(i, _):
        onesv[pl.ds(i * 16, 16)] = jnp.ones((16,), jnp.float32)
        return 0

    lax.fori_loop(0, CH // 16, ones_body, 0)

    pltpu.sync_copy(zerov, degsp.at[pl.ds(s * RPT, RPT)])
    wid = c * NS + s
    pltpu.sync_copy(dst_hbm.at[wid], dstv)
    plsc.subcore_barrier()

    def body(i, _):
        pltpu.sync_copy(onesv, degsp.at[dstv.at[i]], add=True)
        return 0

    lax.fori_loop(0, NCHUNK, body, 0)
    plsc.subcore_barrier()

    pltpu.sync_copy(degsp.at[pl.ds(s * RPT, RPT)], degbuf)

    def wide_body(n, _):
        splat = plsc.load_gather(degbuf, [jnp.full((16,), n, jnp.int32)])
        row = n >> 1
        colb = (n & 1) * 64
        for r in range(4):
            widev[row, pl.ds(colb + r * 16, 16)] = splat
        return 0

    lax.fori_loop(0, RPT, wide_body, 0)
    pltpu.sync_copy(widev, degw_hbm.at[c, pl.ds(s * (RPT // 2), RPT // 2)])


_deg_call = pl.kernel(
    _deg_body,
    out_type=jax.ShapeDtypeStruct((NC, NH, 128), jnp.float32),
    mesh=plsc.VectorSubcoreMesh(**_MESH),
    compiler_params=_SC_PARAMS,
    scratch_types=[
        pltpu.VMEM((NCHUNK, CH), jnp.int32),
        pltpu.VMEM((CH,), jnp.float32),
        pltpu.VMEM((RPT,), jnp.float32),
        pltpu.VMEM((RPT,), jnp.float32),
        pltpu.VMEM((RPT // 2, 128), jnp.float32),
        pltpu.VMEM_SHARED((NPAD,), jnp.float32),
    ],
)


_NBUF = 4


def _agg_body(y_hbm, src_hbm, dst_hbm, out_hbm, srcv, dstv,
              buf0, buf1, buf2, buf3, ysp, tsp,
              g0, g1, g2, g3, s0, s1, s2, s3):
    c = lax.axis_index("c")
    s = lax.axis_index("s")
    wid = c * NS + s
    bufs = (buf0, buf1, buf2, buf3)
    gsem = (g0, g1, g2, g3)
    ssem = (s0, s1, s2, s3)

    pltpu.sync_copy(y_hbm.at[pl.ds(s * RPT, RPT)], ysp.at[pl.ds(s * RPT, RPT)])
    pltpu.sync_copy(y_hbm.at[pl.ds(s * RPT, RPT)], tsp.at[pl.ds(s * RPT, RPT)])
    pltpu.sync_copy(src_hbm.at[wid], srcv)
    pltpu.sync_copy(dst_hbm.at[wid], dstv)
    plsc.subcore_barrier()

    for j in range(_NBUF):
        pltpu.async_copy(ysp.at[srcv.at[j]], bufs[j], gsem[j])

    def body(i, _):
        scat = []
        for j in range(_NBUF):
            cch = _NBUF * i + j
            pltpu.make_async_copy(ysp.at[srcv.at[cch]], bufs[j],
                                  gsem[j]).wait()
            scat.append(pltpu.async_copy(bufs[j], tsp.at[dstv.at[cch]],
                                         ssem[j], add=True))
        for j in range(_NBUF):
            nch = _NBUF * (i + 1) + j
            scat[j].wait()
            pltpu.async_copy(ysp.at[srcv.at[nch]], bufs[j], gsem[j])
        return 0

    lax.fori_loop(0, NCHUNK // _NBUF - 1, body, 0)
    last_pref = _NBUF * (NCHUNK // _NBUF)
    for j in range(_NBUF):
        cch = last_pref - _NBUF + j
        pltpu.make_async_copy(ysp.at[srcv.at[cch]], bufs[j], gsem[j]).wait()
        pltpu.sync_copy(bufs[j], tsp.at[dstv.at[cch]], add=True)
    for cch in range(last_pref, NCHUNK):
        pltpu.async_copy(ysp.at[srcv.at[cch]], bufs[0], gsem[0]).wait()
        pltpu.sync_copy(bufs[0], tsp.at[dstv.at[cch]], add=True)
    plsc.subcore_barrier()

    pltpu.sync_copy(tsp.at[pl.ds(s * RPT, RPT)],
                    out_hbm.at[c, pl.ds(s * RPT, RPT)])


def _make_agg(d):
    return pl.kernel(
        _agg_body,
        out_type=jax.ShapeDtypeStruct((NC, NPAD, d), jnp.float32),
        mesh=plsc.VectorSubcoreMesh(**_MESH),
        compiler_params=_SC_PARAMS,
        scratch_types=[
            pltpu.VMEM((NCHUNK, CH), jnp.int32),
            pltpu.VMEM((NCHUNK, CH), jnp.int32),
        ] + [pltpu.VMEM((CH, d), jnp.float32) for _ in range(_NBUF)] + [
            pltpu.VMEM_SHARED((NPAD, d), jnp.float32),
            pltpu.VMEM_SHARED((NPAD, d), jnp.float32),
        ] + [pltpu.SemaphoreType.DMA for _ in range(2 * _NBUF)],
    )


_agg64 = _make_agg(64)
_agg32 = _make_agg(32)


_BLK = 512
_NBLK = NH // _BLK


def _tc_first_body(x_ref, w_ref, degw_ref, y_ref, dinv_ref):
    dinv = lax.rsqrt(1.0 + degw_ref[0] + degw_ref[1])
    y_ref[...] = jnp.dot(x_ref[...], w_ref[...],
                         preferred_element_type=jnp.float32) * dinv
    dinv_ref[...] = dinv


def _tc_first(x_pair, w1_2, degw):
    return pl.pallas_call(
        _tc_first_body,
        grid=(_NBLK,),
        in_specs=[
            pl.BlockSpec((_BLK, 256), lambda i: (i, 0)),
            pl.BlockSpec((256, 128), lambda i: (0, 0)),
            pl.BlockSpec((NC, _BLK, 128), lambda i: (0, i, 0)),
        ],
        out_specs=[
            pl.BlockSpec((_BLK, 128), lambda i: (i, 0)),
            pl.BlockSpec((_BLK, 128), lambda i: (i, 0)),
        ],
        out_shape=[
            jax.ShapeDtypeStruct((NH, 128), jnp.float32),
            jax.ShapeDtypeStruct((NH, 128), jnp.float32),
        ],
    )(x_pair, w1_2, degw)


def _dinv32(dv):
    return jnp.concatenate([dv[:, 0:32], dv[:, 64:96]], axis=1)


def _tc_mid_body(t_ref, y_ref, dinv_ref, b_ref, w_ref, out_ref, *, d_out):
    dinv = dinv_ref[...]
    h = jnp.maximum(dinv * (t_ref[0] + t_ref[1] - y_ref[...]) + b_ref[...],
                    0.0)
    scale = dinv if d_out == 128 else _dinv32(dinv)
    out_ref[...] = jnp.dot(h, w_ref[...],
                           preferred_element_type=jnp.float32) * scale


def _tc_mid(t, y, dinv, b, w, d_out):
    return pl.pallas_call(
        functools.partial(_tc_mid_body, d_out=d_out),
        grid=(_NBLK,),
        in_specs=[
            pl.BlockSpec((NC, _BLK, 128), lambda i: (0, i, 0)),
            pl.BlockSpec((_BLK, 128), lambda i: (i, 0)),
            pl.BlockSpec((_BLK, 128), lambda i: (i, 0)),
            pl.BlockSpec((1, 128), lambda i: (0, 0)),
            pl.BlockSpec((128, d_out), lambda i: (0, 0)),
        ],
        out_specs=pl.BlockSpec((_BLK, d_out), lambda i: (i, 0)),
        out_shape=jax.ShapeDtypeStruct((NH, d_out), jnp.float32),
    )(t, y, dinv, b, w)


def _tc_last_body(t_ref, y_ref, dinv_ref, b_ref, out_ref):
    out_ref[...] = (_dinv32(dinv_ref[...]) * (t_ref[0] + t_ref[1] - y_ref[...])
                    + b_ref[...])


def _tc_last(t, y, dinv, b):
    return pl.pallas_call(
        _tc_last_body,
        grid=(_NBLK,),
        in_specs=[
            pl.BlockSpec((NC, _BLK, 64), lambda i: (0, i, 0)),
            pl.BlockSpec((_BLK, 64), lambda i: (i, 0)),
            pl.BlockSpec((_BLK, 128), lambda i: (i, 0)),
            pl.BlockSpec((1, 64), lambda i: (0, 0)),
        ],
        out_specs=pl.BlockSpec((_BLK, 64), lambda i: (i, 0)),
        out_shape=jax.ShapeDtypeStruct((NH, 64), jnp.float32),
    )(t, y, dinv, b)


def _blockdiag2(w):
    d_in, d_out = w.shape
    z = jnp.zeros((2 * d_in, 2 * d_out), jnp.float32)
    return z.at[:d_in, :d_out].set(w).at[d_in:, d_out:].set(w)


def kernel(x, edge_index, W1, b1, W2, b2, W3, b3, W4, b4):
    src = edge_index[0].astype(jnp.int32)
    dst = edge_index[1].astype(jnp.int32)
    src3 = src.reshape(NW, NCHUNK, CH)
    dst3 = dst.reshape(NW, NCHUNK, CH)

    x_pair = jnp.pad(x, ((0, NPAD - N), (0, 0))).reshape(NH, 256)
    w1_2 = _blockdiag2(W1)
    w2_2 = _blockdiag2(W2)
    w3_2 = _blockdiag2(W3)
    w4_2 = _blockdiag2(W4)
    b1p = jnp.concatenate([b1, b1])[None, :]
    b2p = jnp.concatenate([b2, b2])[None, :]
    b3p = jnp.concatenate([b3, b3])[None, :]
    b4p = jnp.concatenate([b4, b4])[None, :]

    degw = _deg_call(dst3)
    y1, dinvw = _tc_first(x_pair, w1_2, degw)

    t = _agg64(y1.reshape(NPAD, 64), src3, dst3)
    y2 = _tc_mid(t.reshape(NC, NH, 128), y1, dinvw, b1p, w2_2, 128)
    t = _agg64(y2.reshape(NPAD, 64), src3, dst3)
    y3 = _tc_mid(t.reshape(NC, NH, 128), y2, dinvw, b2p, w3_2, 128)
    t = _agg64(y3.reshape(NPAD, 64), src3, dst3)
    y4 = _tc_mid(t.reshape(NC, NH, 128), y3, dinvw, b3p, w4_2, 64)
    t = _agg32(y4.reshape(NPAD, 32), src3, dst3)
    out = _tc_last(t.reshape(NC, NH, 64), y4, dinvw, b4p)

    return out.reshape(NPAD, 32)[:N]
